# Initial kernel scaffold; baseline (speedup 1.0000x reference)
import jax
import jax.numpy as jnp
from jax.experimental import pallas as pl


def kernel(input, input_mask, embed):
    raise NotImplementedError("write your pallas kernel here")


# speedup vs baseline: 1.9080x; 1.9080x over previous
import functools

import jax
import jax.numpy as jnp
from jax.experimental import pallas as pl
from jax.experimental.pallas import tpu as pltpu

DIM = 64
N_EMBED = 1024
N_ROWS = 16384
BLK = 2048
NBLK = N_ROWS // BLK


def _vq_body(x_ref, m_ref, e_ref, outq_ref, idx_ref, scal_ref,
             counts_ref, acc_ref):
    i = pl.program_id(0)

    @pl.when(i == 0)
    def _init():
        counts_ref[...] = jnp.zeros_like(counts_ref)
        acc_ref[0] = 0.0
        acc_ref[1] = 0.0

    x = x_ref[...]
    m = m_ref[...]
    e = e_ref[...]

    xe = jnp.dot(x, e, preferred_element_type=jnp.float32)
    x2 = jnp.sum(x * x, axis=1, keepdims=True)
    e2 = jnp.sum(e * e, axis=0, keepdims=True)
    dist = x2 - 2.0 * xe + e2
    idx = jnp.argmax(-dist, axis=1)

    onehot = (jax.lax.broadcasted_iota(jnp.int32, (BLK, N_EMBED), 1)
              == idx[:, None]).astype(jnp.float32)
    quant = jax.lax.dot_general(onehot, e, (((1,), (1,)), ((), ())),
                                preferred_element_type=jnp.float32)

    qm = quant * m
    xm = x * m
    outq_ref[...] = xm + (qm - xm)
    idx_ref[...] = idx[:, None]

    counts_ref[...] += jnp.sum(onehot * m, axis=0, keepdims=True)
    d = qm - xm
    acc_ref[0] += jnp.sum(d * d)
    acc_ref[1] += jnp.sum(m)

    @pl.when(i == NBLK - 1)
    def _fin():
        diff = acc_ref[0] / (N_ROWS * DIM)
        sel = counts_ref[...] / acc_ref[1]
        eff = 1.0 / jnp.sum(sel * sel)
        scal_ref[0] = diff
        scal_ref[1] = eff


@functools.partial(jax.jit, static_argnames=())
def kernel(input, input_mask, embed):
    T, B, dim = input.shape
    x = input.reshape(N_ROWS, DIM)
    m = input_mask.reshape(N_ROWS, 1).astype(jnp.float32)

    outq, idx, scal = pl.pallas_call(
        _vq_body,
        grid=(NBLK,),
        in_specs=[
            pl.BlockSpec((BLK, DIM), lambda i: (i, 0)),
            pl.BlockSpec((BLK, 1), lambda i: (i, 0)),
            pl.BlockSpec((DIM, N_EMBED), lambda i: (0, 0)),
        ],
        out_specs=[
            pl.BlockSpec((BLK, DIM), lambda i: (i, 0)),
            pl.BlockSpec((BLK, 1), lambda i: (i, 0)),
            pl.BlockSpec(memory_space=pltpu.SMEM),
        ],
        out_shape=[
            jax.ShapeDtypeStruct((N_ROWS, DIM), jnp.float32),
            jax.ShapeDtypeStruct((N_ROWS, 1), jnp.int32),
            jax.ShapeDtypeStruct((2,), jnp.float32),
        ],
        scratch_shapes=[
            pltpu.VMEM((1, N_EMBED), jnp.float32),
            pltpu.SMEM((2,), jnp.float32),
        ],
    )(x, m, embed)

    quantize_out = outq.reshape(T, B, dim)
    embed_ind = idx.reshape(T, B)
    return (quantize_out, scal[0], embed_ind, scal[1])
